# Initial kernel scaffold; baseline (speedup 1.0000x reference)
import jax
import jax.numpy as jnp
from jax.experimental import pallas as pl


def kernel(x_producer, x_injector, edge_index, proj_W, proj_b, sage_Wl, sage_bl, sage_Wr, gru_Wih, gru_Whh, gru_bih, gru_bhh, gru_oW, gru_ob, fus_W1, fus_b1, fus_W2, fus_b2, head_W1, head_b1, head_W2, head_b2):
    raise NotImplementedError("write your pallas kernel here")


# speedup vs baseline: 1.3418x; 1.3418x over previous
import functools

import jax
import jax.numpy as jnp
from jax import lax
from jax.experimental import pallas as pl
from jax.experimental.pallas import tpu as pltpu
from jax.experimental.pallas import tpu_sc as plsc

G = 2
NT = 2
L = 2
R = 2
T = 6
NP = 10000
D = 128
H = 128
HOR = 12
E = 160000

NPAD = 10240
NC = 2
NS = 16
NTILE = NC * NS
CH = 128
NCH = 40
EPT = CH * NCH
EPAD = NTILE * EPT
RPT = NPAD // NS

def _sc_segsum_body(xp, xi, src0, dst0, src1, dst1, agg_i, agg_p,
                    srcv, dstv0, dstv1, rowbuf, zbuf, acc, sem):
    c = lax.axis_index("c")
    s = lax.axis_index("s")
    wid = s * NC + c

    def _z(r, carry):
        for k in range(H // 16):
            zbuf[r, pl.ds(k * 16, 16)] = jnp.zeros((16,), jnp.float32)
        return carry

    lax.fori_loop(0, CH, _z, 0)
    pltpu.sync_copy(dst0.at[wid], dstv0)
    pltpu.sync_copy(dst1.at[wid], dstv1)

    def per_t(t, carry):
        for rel in range(2):
            src = (src0, src1)[rel]
            x = (xp, xi)[rel]
            dstv = (dstv0, dstv1)[rel]
            out = (agg_i, agg_p)[rel]
            pltpu.sync_copy(src.at[t, wid], srcv)
            for k in range(RPT // CH):
                pltpu.sync_copy(zbuf, acc.at[pl.ds(s * RPT + k * CH, CH)])
            plsc.subcore_barrier()

            def per_chunk(j, inner):
                pltpu.async_copy(x.at[srcv.at[j]], rowbuf, sem).wait()
                pltpu.sync_copy(rowbuf, acc.at[dstv.at[j]], add=True)
                return inner

            lax.fori_loop(0, NCH, per_chunk, 0)
            plsc.subcore_barrier()
            pltpu.sync_copy(acc.at[pl.ds(s * RPT, RPT)],
                            out.at[c, t, pl.ds(s * RPT, RPT)])
        return carry

    lax.fori_loop(0, T, per_t, 0)


def _sc_counts_body(dst, cnt, dstv, ones, zeros, acc):
    c = lax.axis_index("c")
    s = lax.axis_index("s")
    wid = s * NC + c

    def _f(r, carry):
        for k in range(H // 16):
            ones[r, pl.ds(k * 16, 16)] = jnp.ones((16,), jnp.float32)
            zeros[r, pl.ds(k * 16, 16)] = jnp.zeros((16,), jnp.float32)
        return carry

    lax.fori_loop(0, CH, _f, 0)
    pltpu.sync_copy(dst.at[wid], dstv)
    for k in range(RPT // CH):
        pltpu.sync_copy(zeros, acc.at[pl.ds(s * RPT + k * CH, CH)])
    plsc.subcore_barrier()

    def per_chunk(j, inner):
        pltpu.sync_copy(ones, acc.at[dstv.at[j]], add=True)
        return inner

    lax.fori_loop(0, NCH, per_chunk, 0)
    plsc.subcore_barrier()
    pltpu.sync_copy(acc.at[pl.ds(s * RPT, RPT)], cnt.at[c, pl.ds(s * RPT, RPT)])


@functools.cache
def _sc_kernels():
    mesh = plsc.VectorSubcoreMesh(core_axis_name="c", subcore_axis_name="s")
    segsum = functools.partial(
        pl.kernel,
        mesh=mesh,
        out_type=(
            jax.ShapeDtypeStruct((NC, T, NPAD, H), jnp.float32),
            jax.ShapeDtypeStruct((NC, T, NPAD, H), jnp.float32),
        ),
        scratch_types=[
            pltpu.VMEM((NCH, CH), jnp.int32),
            pltpu.VMEM((NCH, CH), jnp.int32),
            pltpu.VMEM((NCH, CH), jnp.int32),
            pltpu.VMEM((CH, H), jnp.float32),
            pltpu.VMEM((CH, H), jnp.float32),
            pltpu.VMEM_SHARED((NPAD, H), jnp.float32),
            pltpu.SemaphoreType.DMA,
        ],
    )(_sc_segsum_body)
    counts = functools.partial(
        pl.kernel,
        mesh=mesh,
        out_type=jax.ShapeDtypeStruct((NC, NPAD, H), jnp.float32),
        scratch_types=[
            pltpu.VMEM((NCH, CH), jnp.int32),
            pltpu.VMEM((CH, H), jnp.float32),
            pltpu.VMEM((CH, H), jnp.float32),
            pltpu.VMEM_SHARED((NPAD, H), jnp.float32),
        ],
    )(_sc_counts_body)
    return segsum, counts


def _sc_segsum(xp, xi, src0, dst0, src1, dst1):
    return _sc_kernels()[0](xp, xi, src0, dst0, src1, dst1)


def _sc_counts(dst):
    return _sc_kernels()[1](dst)


def _mm(x, w, b, relu, bm=512):
    M, K = x.shape
    N = w.shape[1]

    def kern(x_ref, w_ref, b_ref, o_ref):
        o = jnp.dot(x_ref[...], w_ref[...],
                    preferred_element_type=jnp.float32) + b_ref[...]
        o_ref[...] = jnp.maximum(o, 0.0) if relu else o

    return pl.pallas_call(
        kern, grid=(M // bm,),
        in_specs=[pl.BlockSpec((bm, K), lambda i: (i, 0)),
                  pl.BlockSpec((K, N), lambda i: (0, 0)),
                  pl.BlockSpec((1, N), lambda i: (0, 0))],
        out_specs=pl.BlockSpec((bm, N), lambda i: (i, 0)),
        out_shape=jax.ShapeDtypeStruct((M, N), jnp.float32),
    )(x, w, b.reshape(1, N))


def _sage_layer(p, invc, x, wl, bl, wr, bm=512):
    M = x.shape[0]

    def kern(p0_ref, p1_ref, ic_ref, x_ref, wl_ref, bl_ref, wr_ref, o_ref):
        agg = (p0_ref[...] + p1_ref[...]) * ic_ref[...]
        o = (jnp.dot(agg, wl_ref[...], preferred_element_type=jnp.float32)
             + jnp.dot(x_ref[...], wr_ref[...], preferred_element_type=jnp.float32)
             + bl_ref[...])
        o_ref[...] = jnp.maximum(o, 0.0)

    return pl.pallas_call(
        kern, grid=(M // bm,),
        in_specs=[pl.BlockSpec((bm, H), lambda i: (i, 0)),
                  pl.BlockSpec((bm, H), lambda i: (i, 0)),
                  pl.BlockSpec((bm, 1), lambda i: (i, 0)),
                  pl.BlockSpec((bm, H), lambda i: (i, 0)),
                  pl.BlockSpec((H, H), lambda i: (0, 0)),
                  pl.BlockSpec((1, H), lambda i: (0, 0)),
                  pl.BlockSpec((H, H), lambda i: (0, 0))],
        out_specs=pl.BlockSpec((bm, H), lambda i: (i, 0)),
        out_shape=jax.ShapeDtypeStruct((M, H), jnp.float32),
    )(p[0], p[1], invc, x, wl, bl.reshape(1, H), wr)


def _gru_emb(gi, whh, bhh, ow, ob, bm=512):
    def kern(gi_ref, whh_ref, bhh_ref, ow_ref, ob_ref, emb_ref, sum_ref):
        i = pl.program_id(0)
        h = jnp.zeros((bm, H), jnp.float32)
        for t in range(T):
            gi_t = gi_ref[t]
            gh = jnp.dot(h, whh_ref[...],
                         preferred_element_type=jnp.float32) + bhh_ref[...]
            r = jax.nn.sigmoid(gi_t[:, :H] + gh[:, :H])
            z = jax.nn.sigmoid(gi_t[:, H:2 * H] + gh[:, H:2 * H])
            n = jnp.tanh(gi_t[:, 2 * H:] + r * gh[:, 2 * H:])
            h = (1.0 - z) * n + z * h
        emb = jnp.dot(h, ow_ref[...], preferred_element_type=jnp.float32) + ob_ref[...]
        emb_ref[...] = emb
        rows = i * bm + lax.broadcasted_iota(jnp.int32, (bm, 1), 0)
        msum = jnp.sum(jnp.where(rows < NP, emb, 0.0), axis=0, keepdims=True)

        @pl.when(i == 0)
        def _():
            sum_ref[...] = jnp.zeros_like(sum_ref)

        sum_ref[...] += msum

    return pl.pallas_call(
        kern, grid=(NPAD // bm,),
        in_specs=[pl.BlockSpec((T, bm, 3 * H), lambda i: (0, i, 0)),
                  pl.BlockSpec((H, 3 * H), lambda i: (0, 0)),
                  pl.BlockSpec((1, 3 * H), lambda i: (0, 0)),
                  pl.BlockSpec((H, H), lambda i: (0, 0)),
                  pl.BlockSpec((1, H), lambda i: (0, 0))],
        out_specs=(pl.BlockSpec((bm, H), lambda i: (i, 0)),
                   pl.BlockSpec((1, H), lambda i: (0, 0))),
        out_shape=(jax.ShapeDtypeStruct((NPAD, H), jnp.float32),
                   jax.ShapeDtypeStruct((1, H), jnp.float32)),
    )(gi, whh, bhh.reshape(1, 3 * H), ow, ob.reshape(1, H))


def _scores(summ8, w1, b1, w2p, b2p):
    def kern(s_ref, w1_ref, b1_ref, w2_ref, b2_ref, o_ref):
        h1 = jnp.maximum(
            jnp.dot(s_ref[...], w1_ref[...],
                    preferred_element_type=jnp.float32) + b1_ref[...], 0.0)
        o_ref[...] = jnp.dot(h1, w2_ref[...],
                             preferred_element_type=jnp.float32) + b2_ref[...]

    return pl.pallas_call(
        kern,
        out_shape=jax.ShapeDtypeStruct((8, H), jnp.float32),
    )(summ8, w1, b1.reshape(1, H), w2p, b2p.reshape(1, H))


def _head(wv, e0, e1, w1, b1, w2p, b2p, bm=512):
    def kern(wv_ref, e0_ref, e1_ref, w1_ref, b1_ref, w2_ref, b2_ref, o_ref):
        fused = wv_ref[0, 0] * e0_ref[...] + wv_ref[0, 1] * e1_ref[...]
        h1 = jnp.maximum(
            jnp.dot(fused, w1_ref[...],
                    preferred_element_type=jnp.float32) + b1_ref[...], 0.0)
        o_ref[...] = jnp.dot(h1, w2_ref[...],
                             preferred_element_type=jnp.float32) + b2_ref[...]

    return pl.pallas_call(
        kern, grid=(NPAD // bm,),
        in_specs=[pl.BlockSpec((8, 128), lambda i: (0, 0)),
                  pl.BlockSpec((bm, H), lambda i: (i, 0)),
                  pl.BlockSpec((bm, H), lambda i: (i, 0)),
                  pl.BlockSpec((H, H), lambda i: (0, 0)),
                  pl.BlockSpec((1, H), lambda i: (0, 0)),
                  pl.BlockSpec((H, 128), lambda i: (0, 0)),
                  pl.BlockSpec((1, 128), lambda i: (0, 0))],
        out_specs=pl.BlockSpec((bm, 128), lambda i: (i, 0)),
        out_shape=jax.ShapeDtypeStruct((NPAD, 128), jnp.float32),
    )(wv, e0, e1, w1, b1.reshape(1, H), w2p, b2p.reshape(1, 128))


def kernel(x_producer, x_injector, edge_index, proj_W, proj_b, sage_Wl, sage_bl,
           sage_Wr, gru_Wih, gru_Whh, gru_bih, gru_bhh, gru_oW, gru_ob,
           fus_W1, fus_b1, fus_W2, fus_b2, head_W1, head_b1, head_W2, head_b2):
    xp = jnp.pad(x_producer, ((0, 0), (0, NPAD - NP), (0, 0))).reshape(T * NPAD, D)
    xi = jnp.pad(x_injector, ((0, 0), (0, NPAD - NP), (0, 0))).reshape(T * NPAD, D)
    ei = edge_index.astype(jnp.int32)
    toff = (jnp.arange(T, dtype=jnp.int32) * NPAD)[:, None, None, None]
    padn = EPAD - E

    def prep(src, dst):
        srcp = jnp.concatenate(
            [src, jnp.full((padn,), NP, jnp.int32)]).reshape(NTILE, NCH, CH)
        dstp = jnp.concatenate(
            [dst, jnp.full((padn,), NP, jnp.int32)]).reshape(NTILE, NCH, CH)
        return srcp[None] + toff, dstp

    edges = []
    for g in range(G):
        per_g = []
        for r in range(R):
            srct, dstp = prep(ei[g, r, 0], ei[g, r, 1])
            cnt = _sc_counts(dstp)
            csum = cnt[0, :, 0] + cnt[1, :, 0]
            ivt = jnp.tile(1.0 / jnp.maximum(csum, 1.0), (T,)).reshape(T * NPAD, 1)
            per_g.append((srct, dstp, ivt))
        edges.append(per_g)

    xp0 = _mm(xp, proj_W[0], proj_b[0], True)
    xi0 = _mm(xi, proj_W[1], proj_b[1], True)

    w2p = jnp.pad(fus_W2, ((0, 0), (0, H - 1)))
    b2p = jnp.pad(fus_b2, (0, H - 1))
    hw2p = jnp.pad(head_W2, ((0, 0), (0, 128 - HOR)))
    hb2p = jnp.pad(head_b2, (0, 128 - HOR))

    embs = []
    for g in range(G):
        hp, hi = xp0, xi0
        srct0, dstp0, ivt0 = edges[g][0]
        srct1, dstp1, ivt1 = edges[g][1]
        for l in range(L):
            agg_i, agg_p = _sc_segsum(hp, hi, srct0, dstp0, srct1, dstp1)
            agg_i = agg_i.reshape(NC, T * NPAD, H)
            agg_p = agg_p.reshape(NC, T * NPAD, H)
            hi_new = _sage_layer(agg_i, ivt0, hi, sage_Wl[g, l, 0],
                                 sage_bl[g, l, 0], sage_Wr[g, l, 0])
            hp_new = _sage_layer(agg_p, ivt1, hp, sage_Wl[g, l, 1],
                                 sage_bl[g, l, 1], sage_Wr[g, l, 1])
            hp, hi = hp_new, hi_new
        gi_p = _mm(hp, gru_Wih[g, 0], gru_bih[g, 0], False).reshape(T, NPAD, 3 * H)
        gi_i = _mm(hi, gru_Wih[g, 1], gru_bih[g, 1], False).reshape(T, NPAD, 3 * H)
        emb_p, sum_p = _gru_emb(gi_p, gru_Whh[g, 0], gru_bhh[g, 0],
                                gru_oW[g, 0], gru_ob[g, 0])
        _, sum_i = _gru_emb(gi_i, gru_Whh[g, 1], gru_bhh[g, 1],
                            gru_oW[g, 1], gru_ob[g, 1])
        embs.append((emb_p, sum_p, sum_i))

    summ = jnp.stack(
        [(embs[g][1][0] + embs[g][2][0]) / (2.0 * NP) for g in range(G)], axis=0)
    summ8 = jnp.pad(summ, ((0, 8 - G), (0, 0)))
    score = _scores(summ8, fus_W1, fus_b1, w2p, b2p)[:G, 0]
    w = jax.nn.softmax(score)
    wv = jnp.pad(w.reshape(1, 2), ((0, 7), (0, 126)))
    ho = _head(wv, embs[0][0], embs[1][0], head_W1, head_b1, hw2p, hb2p)
    return ho[:NP, :HOR]


# speedup vs baseline: 1.4854x; 1.1070x over previous
import functools

import jax
import jax.numpy as jnp
from jax import lax
from jax.experimental import pallas as pl
from jax.experimental.pallas import tpu as pltpu
from jax.experimental.pallas import tpu_sc as plsc

G = 2
NT = 2
L = 2
R = 2
T = 6
NP = 10000
D = 128
H = 128
HOR = 12
E = 160000

NPAD = 10240
NC = 2
NS = 16
NTILE = NC * NS
CH = 128
NCH = 40
EPT = CH * NCH
EPAD = NTILE * EPT
RPT = NPAD // NS

NBUF = 2
NGRP = NCH // NBUF
ZR = 16


def _sc_segsum_body(xp, xi, src0, dst0, src1, dst1, agg_i, agg_p,
                    srcv, dstv, b0, b1, zbuf, acc, s0, s1):
    bufs = (b0, b1)
    sems = (s0, s1)
    c = lax.axis_index("c")
    s = lax.axis_index("s")
    wid = s * NC + c

    def _z(r, carry):
        for k in range(H // 16):
            zbuf[r, pl.ds(k * 16, 16)] = jnp.zeros((16,), jnp.float32)
        return carry

    lax.fori_loop(0, ZR, _z, 0)

    def per_t(t, carry):
        for rel in range(2):
            src = (src0, src1)[rel]
            x = (xp, xi)[rel]
            dst = (dst0, dst1)[rel]
            out = (agg_i, agg_p)[rel]
            pltpu.sync_copy(src.at[t, wid], srcv)
            pltpu.sync_copy(dst.at[wid], dstv)
            for k in range(RPT // ZR):
                pltpu.sync_copy(zbuf, acc.at[pl.ds(s * RPT + k * ZR, ZR)])
            plsc.subcore_barrier()
            for b in range(NBUF):
                pltpu.async_copy(x.at[srcv.at[b]], bufs[b], sems[b])

            def per_grp(grp, inner):
                for b in range(NBUF):
                    j = grp * NBUF + b
                    pltpu.make_async_copy(x.at[srcv.at[j]], bufs[b],
                                          sems[b]).wait()
                    pltpu.sync_copy(bufs[b], acc.at[dstv.at[j]], add=True)

                    @pl.when(grp < NGRP - 1)
                    def _():
                        pltpu.async_copy(x.at[srcv.at[j + NBUF]], bufs[b],
                                         sems[b])
                return inner

            lax.fori_loop(0, NGRP, per_grp, 0)
            plsc.subcore_barrier()
            pltpu.sync_copy(acc.at[pl.ds(s * RPT, RPT)],
                            out.at[c, t, pl.ds(s * RPT, RPT)])
        return carry

    lax.fori_loop(0, T, per_t, 0)


def _sc_counts_body(dst, cnt, dstv, ones, zeros, acc):
    c = lax.axis_index("c")
    s = lax.axis_index("s")
    wid = s * NC + c

    def _f(r, carry):
        for k in range(H // 16):
            ones[r, pl.ds(k * 16, 16)] = jnp.ones((16,), jnp.float32)
        return carry

    lax.fori_loop(0, CH, _f, 0)

    def _g(r, carry):
        for k in range(H // 16):
            zeros[r, pl.ds(k * 16, 16)] = jnp.zeros((16,), jnp.float32)
        return carry

    lax.fori_loop(0, ZR, _g, 0)
    pltpu.sync_copy(dst.at[wid], dstv)
    for k in range(RPT // ZR):
        pltpu.sync_copy(zeros, acc.at[pl.ds(s * RPT + k * ZR, ZR)])
    plsc.subcore_barrier()

    def per_chunk(j, inner):
        pltpu.sync_copy(ones, acc.at[dstv.at[j]], add=True)
        return inner

    lax.fori_loop(0, NCH, per_chunk, 0)
    plsc.subcore_barrier()
    pltpu.sync_copy(acc.at[pl.ds(s * RPT, RPT)], cnt.at[c, pl.ds(s * RPT, RPT)])


@functools.cache
def _sc_kernels():
    mesh = plsc.VectorSubcoreMesh(core_axis_name="c", subcore_axis_name="s")
    segsum = functools.partial(
        pl.kernel,
        mesh=mesh,
        out_type=(
            jax.ShapeDtypeStruct((NC, T, NPAD, H), jnp.float32),
            jax.ShapeDtypeStruct((NC, T, NPAD, H), jnp.float32),
        ),
        scratch_types=[
            pltpu.VMEM((NCH, CH), jnp.int32),
            pltpu.VMEM((NCH, CH), jnp.int32),
            pltpu.VMEM((CH, H), jnp.float32),
            pltpu.VMEM((CH, H), jnp.float32),
            pltpu.VMEM((ZR, H), jnp.float32),
            pltpu.VMEM_SHARED((NPAD, H), jnp.float32),
            pltpu.SemaphoreType.DMA,
            pltpu.SemaphoreType.DMA,
        ],
    )(_sc_segsum_body)
    counts = functools.partial(
        pl.kernel,
        mesh=mesh,
        out_type=jax.ShapeDtypeStruct((NC, NPAD, H), jnp.float32),
        scratch_types=[
            pltpu.VMEM((NCH, CH), jnp.int32),
            pltpu.VMEM((CH, H), jnp.float32),
            pltpu.VMEM((ZR, H), jnp.float32),
            pltpu.VMEM_SHARED((NPAD, H), jnp.float32),
        ],
    )(_sc_counts_body)
    return segsum, counts


def _sc_segsum(xp, xi, src0, dst0, src1, dst1):
    return _sc_kernels()[0](xp, xi, src0, dst0, src1, dst1)


def _sc_counts(dst):
    return _sc_kernels()[1](dst)


def _mm(x, w, b, relu, bm=512):
    M, K = x.shape
    N = w.shape[1]

    def kern(x_ref, w_ref, b_ref, o_ref):
        o = jnp.dot(x_ref[...], w_ref[...],
                    preferred_element_type=jnp.float32) + b_ref[...]
        o_ref[...] = jnp.maximum(o, 0.0) if relu else o

    return pl.pallas_call(
        kern, grid=(M // bm,),
        in_specs=[pl.BlockSpec((bm, K), lambda i: (i, 0)),
                  pl.BlockSpec((K, N), lambda i: (0, 0)),
                  pl.BlockSpec((1, N), lambda i: (0, 0))],
        out_specs=pl.BlockSpec((bm, N), lambda i: (i, 0)),
        out_shape=jax.ShapeDtypeStruct((M, N), jnp.float32),
    )(x, w, b.reshape(1, N))


def _sage_layer(p, invc, x, wl, bl, wr, bm=512):
    M = x.shape[0]

    def kern(p0_ref, p1_ref, ic_ref, x_ref, wl_ref, bl_ref, wr_ref, o_ref):
        agg = (p0_ref[...] + p1_ref[...]) * ic_ref[...]
        o = (jnp.dot(agg, wl_ref[...], preferred_element_type=jnp.float32)
             + jnp.dot(x_ref[...], wr_ref[...], preferred_element_type=jnp.float32)
             + bl_ref[...])
        o_ref[...] = jnp.maximum(o, 0.0)

    return pl.pallas_call(
        kern, grid=(M // bm,),
        in_specs=[pl.BlockSpec((bm, H), lambda i: (i, 0)),
                  pl.BlockSpec((bm, H), lambda i: (i, 0)),
                  pl.BlockSpec((bm, 1), lambda i: (i, 0)),
                  pl.BlockSpec((bm, H), lambda i: (i, 0)),
                  pl.BlockSpec((H, H), lambda i: (0, 0)),
                  pl.BlockSpec((1, H), lambda i: (0, 0)),
                  pl.BlockSpec((H, H), lambda i: (0, 0))],
        out_specs=pl.BlockSpec((bm, H), lambda i: (i, 0)),
        out_shape=jax.ShapeDtypeStruct((M, H), jnp.float32),
    )(p[0], p[1], invc, x, wl, bl.reshape(1, H), wr)


def _gru_emb(gi, whh, bhh, ow, ob, bm=512):
    def kern(gi_ref, whh_ref, bhh_ref, ow_ref, ob_ref, emb_ref, sum_ref):
        i = pl.program_id(0)
        h = jnp.zeros((bm, H), jnp.float32)
        for t in range(T):
            gi_t = gi_ref[t]
            gh = jnp.dot(h, whh_ref[...],
                         preferred_element_type=jnp.float32) + bhh_ref[...]
            r = jax.nn.sigmoid(gi_t[:, :H] + gh[:, :H])
            z = jax.nn.sigmoid(gi_t[:, H:2 * H] + gh[:, H:2 * H])
            n = jnp.tanh(gi_t[:, 2 * H:] + r * gh[:, 2 * H:])
            h = (1.0 - z) * n + z * h
        emb = jnp.dot(h, ow_ref[...], preferred_element_type=jnp.float32) + ob_ref[...]
        emb_ref[...] = emb
        rows = i * bm + lax.broadcasted_iota(jnp.int32, (bm, 1), 0)
        msum = jnp.sum(jnp.where(rows < NP, emb, 0.0), axis=0, keepdims=True)

        @pl.when(i == 0)
        def _():
            sum_ref[...] = jnp.zeros_like(sum_ref)

        sum_ref[...] += msum

    return pl.pallas_call(
        kern, grid=(NPAD // bm,),
        in_specs=[pl.BlockSpec((T, bm, 3 * H), lambda i: (0, i, 0)),
                  pl.BlockSpec((H, 3 * H), lambda i: (0, 0)),
                  pl.BlockSpec((1, 3 * H), lambda i: (0, 0)),
                  pl.BlockSpec((H, H), lambda i: (0, 0)),
                  pl.BlockSpec((1, H), lambda i: (0, 0))],
        out_specs=(pl.BlockSpec((bm, H), lambda i: (i, 0)),
                   pl.BlockSpec((1, H), lambda i: (0, 0))),
        out_shape=(jax.ShapeDtypeStruct((NPAD, H), jnp.float32),
                   jax.ShapeDtypeStruct((1, H), jnp.float32)),
    )(gi, whh, bhh.reshape(1, 3 * H), ow, ob.reshape(1, H))


def _scores(summ8, w1, b1, w2p, b2p):
    def kern(s_ref, w1_ref, b1_ref, w2_ref, b2_ref, o_ref):
        h1 = jnp.maximum(
            jnp.dot(s_ref[...], w1_ref[...],
                    preferred_element_type=jnp.float32) + b1_ref[...], 0.0)
        o_ref[...] = jnp.dot(h1, w2_ref[...],
                             preferred_element_type=jnp.float32) + b2_ref[...]

    return pl.pallas_call(
        kern,
        out_shape=jax.ShapeDtypeStruct((8, H), jnp.float32),
    )(summ8, w1, b1.reshape(1, H), w2p, b2p.reshape(1, H))


def _head(wv, e0, e1, w1, b1, w2p, b2p, bm=512):
    def kern(wv_ref, e0_ref, e1_ref, w1_ref, b1_ref, w2_ref, b2_ref, o_ref):
        fused = wv_ref[0, 0] * e0_ref[...] + wv_ref[0, 1] * e1_ref[...]
        h1 = jnp.maximum(
            jnp.dot(fused, w1_ref[...],
                    preferred_element_type=jnp.float32) + b1_ref[...], 0.0)
        o_ref[...] = jnp.dot(h1, w2_ref[...],
                             preferred_element_type=jnp.float32) + b2_ref[...]

    return pl.pallas_call(
        kern, grid=(NPAD // bm,),
        in_specs=[pl.BlockSpec((8, 128), lambda i: (0, 0)),
                  pl.BlockSpec((bm, H), lambda i: (i, 0)),
                  pl.BlockSpec((bm, H), lambda i: (i, 0)),
                  pl.BlockSpec((H, H), lambda i: (0, 0)),
                  pl.BlockSpec((1, H), lambda i: (0, 0)),
                  pl.BlockSpec((H, 128), lambda i: (0, 0)),
                  pl.BlockSpec((1, 128), lambda i: (0, 0))],
        out_specs=pl.BlockSpec((bm, 128), lambda i: (i, 0)),
        out_shape=jax.ShapeDtypeStruct((NPAD, 128), jnp.float32),
    )(wv, e0, e1, w1, b1.reshape(1, H), w2p, b2p.reshape(1, 128))


def kernel(x_producer, x_injector, edge_index, proj_W, proj_b, sage_Wl, sage_bl,
           sage_Wr, gru_Wih, gru_Whh, gru_bih, gru_bhh, gru_oW, gru_ob,
           fus_W1, fus_b1, fus_W2, fus_b2, head_W1, head_b1, head_W2, head_b2):
    xp = jnp.pad(x_producer, ((0, 0), (0, NPAD - NP), (0, 0))).reshape(T * NPAD, D)
    xi = jnp.pad(x_injector, ((0, 0), (0, NPAD - NP), (0, 0))).reshape(T * NPAD, D)
    ei = edge_index.astype(jnp.int32)
    toff = (jnp.arange(T, dtype=jnp.int32) * NPAD)[:, None, None, None]
    padn = EPAD - E

    def prep(src, dst):
        srcp = jnp.concatenate(
            [src, jnp.full((padn,), NP, jnp.int32)]).reshape(NTILE, NCH, CH)
        dstp = jnp.concatenate(
            [dst, jnp.full((padn,), NP, jnp.int32)]).reshape(NTILE, NCH, CH)
        return srcp[None] + toff, dstp

    edges = []
    for g in range(G):
        per_g = []
        for r in range(R):
            srct, dstp = prep(ei[g, r, 0], ei[g, r, 1])
            cnt = _sc_counts(dstp)
            csum = cnt[0, :, 0] + cnt[1, :, 0]
            ivt = jnp.tile(1.0 / jnp.maximum(csum, 1.0), (T,)).reshape(T * NPAD, 1)
            per_g.append((srct, dstp, ivt))
        edges.append(per_g)

    xp0 = _mm(xp, proj_W[0], proj_b[0], True)
    xi0 = _mm(xi, proj_W[1], proj_b[1], True)

    w2p = jnp.pad(fus_W2, ((0, 0), (0, H - 1)))
    b2p = jnp.pad(fus_b2, (0, H - 1))
    hw2p = jnp.pad(head_W2, ((0, 0), (0, 128 - HOR)))
    hb2p = jnp.pad(head_b2, (0, 128 - HOR))

    embs = []
    for g in range(G):
        hp, hi = xp0, xi0
        srct0, dstp0, ivt0 = edges[g][0]
        srct1, dstp1, ivt1 = edges[g][1]
        for l in range(L):
            agg_i, agg_p = _sc_segsum(hp, hi, srct0, dstp0, srct1, dstp1)
            agg_i = agg_i.reshape(NC, T * NPAD, H)
            agg_p = agg_p.reshape(NC, T * NPAD, H)
            hi_new = _sage_layer(agg_i, ivt0, hi, sage_Wl[g, l, 0],
                                 sage_bl[g, l, 0], sage_Wr[g, l, 0])
            hp_new = _sage_layer(agg_p, ivt1, hp, sage_Wl[g, l, 1],
                                 sage_bl[g, l, 1], sage_Wr[g, l, 1])
            hp, hi = hp_new, hi_new
        gi_p = _mm(hp, gru_Wih[g, 0], gru_bih[g, 0], False).reshape(T, NPAD, 3 * H)
        gi_i = _mm(hi, gru_Wih[g, 1], gru_bih[g, 1], False).reshape(T, NPAD, 3 * H)
        emb_p, sum_p = _gru_emb(gi_p, gru_Whh[g, 0], gru_bhh[g, 0],
                                gru_oW[g, 0], gru_ob[g, 0])
        _, sum_i = _gru_emb(gi_i, gru_Whh[g, 1], gru_bhh[g, 1],
                            gru_oW[g, 1], gru_ob[g, 1])
        embs.append((emb_p, sum_p, sum_i))

    summ = jnp.stack(
        [(embs[g][1][0] + embs[g][2][0]) / (2.0 * NP) for g in range(G)], axis=0)
    summ8 = jnp.pad(summ, ((0, 8 - G), (0, 0)))
    score = _scores(summ8, fus_W1, fus_b1, w2p, b2p)[:G, 0]
    w = jax.nn.softmax(score)
    wv = jnp.pad(w.reshape(1, 2), ((0, 7), (0, 126)))
    ho = _head(wv, embs[0][0], embs[1][0], head_W1, head_b1, hw2p, hb2p)
    return ho[:NP, :HOR]
